# baseline (speedup 1.0000x reference)
import jax
import jax.numpy as jnp
import numpy as np
from jax import lax
from jax.experimental import pallas as pl
from jax.experimental.pallas import tpu as pltpu
from jax.experimental.pallas import tpu_sc as plsc

HID_ = 256
NH_ = 8
DH_ = HID_ // NH_
N_ = 10000
E_ = 160000

NW_ = 32
R_PER_ = 320
NPAD_ = NW_ * R_PER_
EB_ = 24
CH_ = 72
NSLOT_ = 3
EPAD_ = E_ + CH_
NRING_ = 8

_I32MIN = np.int32(-2147483648)


def _qkv_body(ht, hs, wq, wk, wv, bq, bk, bv, qo, kvo):
    dn = (((1,), (1,)), ((), ()))
    scale = jnp.float32(DH_ ** -0.5)
    qo[...] = (lax.dot_general(ht[...], wq[...], dn,
                               preferred_element_type=jnp.float32)
               + bq[...]) * scale
    kvo[:, 0:HID_] = lax.dot_general(hs[...], wk[...], dn,
                                     preferred_element_type=jnp.float32) + bk[...]
    kvo[:, HID_:2 * HID_] = lax.dot_general(hs[...], wv[...], dn,
                                            preferred_element_type=jnp.float32) + bv[...]


def _qkv(ht, hs, wq, wk, wv, bq, bk, bv):
    blk = 80
    grid = NPAD_ // blk
    nin = N_ // blk - 1
    in_spec = pl.BlockSpec((blk, HID_), lambda i: (jnp.minimum(i, nin), 0))
    row_spec = pl.BlockSpec((blk, HID_), lambda i: (i, 0))
    full_spec = pl.BlockSpec((HID_, HID_), lambda i: (0, 0))
    bias_spec = pl.BlockSpec((1, HID_), lambda i: (0, 0))
    return pl.pallas_call(
        _qkv_body,
        grid=(grid,),
        in_specs=[in_spec, in_spec, full_spec, full_spec, full_spec,
                  bias_spec, bias_spec, bias_spec],
        out_specs=[row_spec, pl.BlockSpec((blk, 2 * HID_), lambda i: (i, 0))],
        out_shape=[jax.ShapeDtypeStruct((NPAD_, HID_), jnp.float32),
                   jax.ShapeDtypeStruct((NPAD_, 2 * HID_), jnp.float32)],
    )(ht, hs, wq, wk, wv, bq.reshape(1, HID_), bk.reshape(1, HID_),
      bv.reshape(1, HID_))


def _lane_gather(vec, idx):
    dn = lax.GatherDimensionNumbers(offset_dims=(), collapsed_slice_dims=(0,),
                                    start_index_map=(0,))
    return lax.gather(vec, idx[:, None], dn, (1,),
                      mode=lax.GatherScatterMode.PROMISE_IN_BOUNDS)


def _extract_i32(vec, lane, iota):
    return jnp.max(jnp.where(iota == lane, vec, _I32MIN))


def _edge_body(q_hbm, kv_hbm, rows_hbm, cols_hbm, mv_hbm, bounds_hbm,
               num_hbm, den_hbm,
               bounds_v, rows_ch, cols_ch, mv_ch, kvb, qsl, den_acc, stg,
               sg, si, ss):
    c = lax.axis_index("c")
    s = lax.axis_index("s")
    wid = s * 2 + c
    iota = lax.iota(jnp.int32, 16)
    zero16 = jnp.zeros((16,), jnp.float32)
    NC16 = HID_ // 16

    pltpu.sync_copy(bounds_hbm, bounds_v)
    b0 = bounds_v[pl.ds(0, 16)]
    b1 = bounds_v[pl.ds(16, 16)]
    b2 = bounds_v[pl.ds(32, 16)]

    def bval(w):
        lane = w % 16
        grp = w // 16
        c0 = _extract_i32(b0, lane, iota)
        c1 = _extract_i32(b1, lane, iota)
        c2 = _extract_i32(b2, lane, iota)
        return jnp.where(grp == 0, c0, jnp.where(grp == 1, c1, c2))

    e_lo = bval(wid)
    e_hi = bval(wid + 1)
    r_lo = wid * R_PER_

    pltpu.sync_copy(q_hbm.at[pl.ds(r_lo, R_PER_)], qsl)

    def zrow(i, _):
        den_acc[i, pl.ds(0, 16)] = zero16
        return 0
    lax.fori_loop(0, R_PER_, zrow, 0)

    e0 = (e_lo // 8) * 8
    nch = (e_hi - e0 + CH_ - 1) // CH_
    npair = (nch + 1) // 2

    def fire_idx(ci, p):
        ec = pl.multiple_of(jnp.minimum(e0 + ci * CH_, EPAD_ - CH_), 8)
        pltpu.async_copy(rows_hbm.at[pl.ds(ec, CH_)], rows_ch.at[p], si.at[p])
        pltpu.async_copy(cols_hbm.at[pl.ds(ec, CH_)], cols_ch.at[p], si.at[p])
        pltpu.async_copy(mv_hbm.at[pl.ds(ec, CH_)], mv_ch.at[p], si.at[p])

    def wait_idx(p):
        pltpu.make_async_copy(rows_hbm.at[pl.ds(0, CH_)], rows_ch.at[p],
                              si.at[p]).wait()
        pltpu.make_async_copy(cols_hbm.at[pl.ds(0, CH_)], cols_ch.at[p],
                              si.at[p]).wait()
        pltpu.make_async_copy(mv_hbm.at[pl.ds(0, CH_)], mv_ch.at[p],
                              si.at[p]).wait()

    def fire_kv(p, k, slot):
        pltpu.async_copy(kv_hbm.at[cols_ch.at[p, pl.ds(k * EB_, EB_)]],
                         kvb.at[slot], sg.at[slot])

    def drain_kv(slot):
        pltpu.make_async_copy(kv_hbm.at[pl.ds(0, EB_)], kvb.at[slot],
                              sg.at[slot]).wait()

    def wait_one_store():
        pltpu.make_async_copy(stg.at[0], num_hbm.at[0], ss).wait()

    def do_flush(prev_rl, ns, accs, denv):
        owned = (prev_rl >= 0) & (prev_rl < R_PER_)

        @pl.when(owned & (ns >= NRING_))
        def _():
            wait_one_store()

        @pl.when(owned)
        def _():
            sl = ns % NRING_
            dsafe = jnp.where(denv == 0.0, jnp.float32(1.0), denv)
            for cc in range(NC16):
                stg[sl, pl.ds(cc * 16, 16)] = accs[cc] / dsafe
            pltpu.async_copy(stg.at[sl], num_hbm.at[r_lo + prev_rl], ss)
            psp = prev_rl + iota * 0
            plsc.addupdate_scatter(den_acc, [psp, iota], denv)
        return ns + jnp.where(owned, 1, 0)

    def block_compute(p, k, slot, ci, carry):
        base_blk = e0 + (ci * 3 + k) * EB_
        for off, jlo in ((0, 0), (8, 8)):
            rows16 = rows_ch[p, pl.ds(k * EB_ + off, 16)]
            mv16 = mv_ch[p, pl.ds(k * EB_ + off, 16)]
            base_eg = base_blk + off

            prev_rl, ns, qregs, accs, denv = carry
            rlv = rows16 - r_lo
            shifted = _lane_gather(rlv, jnp.maximum(iota - 1, 0))
            bmask = jnp.where(iota == 0, rlv != (prev_rl + iota * 0),
                              rlv != shifted)

            def wcond(st):
                return st[0] < 16

            def wbody(st, off=off, rlv=rlv, bmask=bmask, mv16=mv16,
                      base_eg=base_eg, slot=slot):
                j, ns, qregs, accs, denv, prl = st
                jsp = iota * 0 + j
                isb = plsc.all_reduce_ffs(bmask & (iota >= jsp))[0] == j
                run_end = plsc.all_reduce_ffs(bmask & (iota > jsp))[0]
                rl_j = _extract_i32(rlv, j, iota)

                def on_b():
                    ns2 = do_flush(prl, ns, accs, denv)
                    rlc = jnp.clip(rl_j, 0, R_PER_ - 1)
                    qn = tuple(qsl[rlc, pl.ds(cc * 16, 16)]
                               for cc in range(NC16))
                    return (ns2,) + qn + tuple(zero16 for _ in range(NC16)) \
                        + (zero16,)

                def no_b():
                    return (ns,) + tuple(qregs) + tuple(accs) + (denv,)

                stc = lax.cond(isb, on_b, no_b)
                ns2 = stc[0]
                qr = stc[1:1 + NC16]
                ac = stc[1 + NC16:1 + 2 * NC16]
                dv = stc[1 + 2 * NC16]

                def edge(j2, ec, off=off, mv16=mv16, base_eg=base_eg,
                         slot=slot, qr=qr):
                    ac, dv = ec
                    jrow = off + j2
                    mvalv = _lane_gather(mv16, iota * 0 + j2)
                    parts = [qr[cc] * kvb[slot, jrow, pl.ds(cc * 16, 16)]
                             for cc in range(NC16)]
                    while len(parts) > 1:
                        parts = [parts[2 * i] + parts[2 * i + 1]
                                 for i in range(len(parts) // 2)]
                    r = parts[0]
                    fold = r + _lane_gather(r, jnp.bitwise_xor(iota, 8))
                    eg = base_eg + j2
                    gate = jnp.where((eg >= e_lo) & (eg < e_hi),
                                     jnp.float32(1.0), jnp.float32(0.0))
                    ex = jnp.exp(fold * mvalv) * gate
                    ac = tuple(ac[cc] + ex * kvb[slot, jrow,
                                                 pl.ds(HID_ + cc * 16, 16)]
                               for cc in range(NC16))
                    return (ac, dv + ex)

                ac, dv = lax.fori_loop(j, run_end, edge, (ac, dv))
                return (run_end, ns2, qr, ac, dv, rl_j)

            j0 = jnp.int32(jlo)
            st = lax.while_loop(wcond, wbody,
                                (j0, ns, tuple(qregs), tuple(accs), denv,
                                 prev_rl))
            carry = (st[5], st[1], st[2], st[3], st[4])
        return carry

    def pair_body(g, carry):
        for p in range(2):
            ci = 2 * g + p
            for k in range(3):
                drain_kv(k)
                carry = block_compute(p, k, k, ci, carry)
                if k == 0:
                    fire_kv(p, 2, 2)
                elif k == 1:
                    wait_idx(1 - p)
                    fire_kv(1 - p, 0, 0)
                else:
                    fire_idx(ci + 2, p)
                    fire_kv(1 - p, 1, 1)
        return carry

    fire_idx(0, 0)
    fire_idx(1, 1)
    wait_idx(0)
    fire_kv(0, 0, 0)
    fire_kv(0, 1, 1)

    init = (jnp.int32(-(2 ** 30)), jnp.int32(0),
            tuple(zero16 for _ in range(NC16)),
            tuple(zero16 for _ in range(NC16)),
            zero16)
    prev_rl, ns, _, accs, denv = lax.fori_loop(0, npair, pair_body, init)
    ns = do_flush(prev_rl, ns, accs, denv)

    drain_kv(0)
    drain_kv(1)
    wait_idx(1)

    def dr(i, _):
        wait_one_store()
        return 0
    lax.fori_loop(0, jnp.minimum(ns, NRING_), dr, 0)

    for cc in range(NC16):
        stg[0, pl.ds(cc * 16, 16)] = zero16

    def zf(i, _):
        dv = den_acc[i, pl.ds(0, 16)]

        @pl.when(dv[0] == 0.0)
        def _():
            pltpu.sync_copy(stg.at[0], num_hbm.at[r_lo + i])
        return 0
    lax.fori_loop(0, R_PER_, zf, 0)

    pltpu.sync_copy(den_acc, den_hbm.at[pl.ds(r_lo, R_PER_)])


def _edge_sc(q, kv, rows_p, cols_p, mv_p, bounds):
    mesh = plsc.VectorSubcoreMesh(core_axis_name="c", subcore_axis_name="s")
    fn = pl.kernel(
        _edge_body,
        out_type=[jax.ShapeDtypeStruct((NPAD_, HID_), jnp.float32),
                  jax.ShapeDtypeStruct((NPAD_, 16), jnp.float32)],
        mesh=mesh,
        scratch_types=[
            pltpu.VMEM((48,), jnp.int32),
            pltpu.VMEM((2, CH_), jnp.int32),
            pltpu.VMEM((2, CH_), jnp.int32),
            pltpu.VMEM((2, CH_), jnp.float32),
            pltpu.VMEM((NSLOT_, EB_, 2 * HID_), jnp.float32),
            pltpu.VMEM((R_PER_, HID_), jnp.float32),
            pltpu.VMEM((R_PER_, 16), jnp.float32),
            pltpu.VMEM((NRING_, HID_), jnp.float32),
            pltpu.SemaphoreType.DMA((NSLOT_,)),
            pltpu.SemaphoreType.DMA((2,)),
            pltpu.SemaphoreType.DMA,
        ],
        compiler_params=pltpu.CompilerParams(needs_layout_passes=False,
                                             use_tc_tiling_on_sc=False),
    )
    return fn(q, kv, rows_p, cols_p, mv_p, bounds)


def _out_body(num, wo, bo, out):
    dn = (((1,), (1,)), ((), ()))
    out[...] = lax.dot_general(num[...], wo[...], dn,
                               preferred_element_type=jnp.float32) + bo[...]


def _outproj(num, wo, bo):
    blk = 200
    grid = N_ // blk
    return pl.pallas_call(
        _out_body,
        grid=(grid,),
        in_specs=[pl.BlockSpec((blk, HID_), lambda i: (i, 0)),
                  pl.BlockSpec((HID_, HID_), lambda i: (0, 0)),
                  pl.BlockSpec((1, HID_), lambda i: (0, 0))],
        out_specs=pl.BlockSpec((blk, HID_), lambda i: (i, 0)),
        out_shape=jax.ShapeDtypeStruct((N_, HID_), jnp.float32),
    )(num, wo, bo.reshape(1, HID_))


@jax.jit
def kernel(h_source, h_target, mask_rows, mask_cols, mask_vals,
           Wq, bq, Wk, bk, Wv, bv, Wo, bo):
    q, kv = _qkv(h_target, h_source, Wq, Wk, Wv, bq, bk, bv)

    rows_i = mask_rows.astype(jnp.int32)
    cols_i = mask_cols.astype(jnp.int32)
    pad_e = EPAD_ - E_
    rows_p = jnp.concatenate([rows_i, jnp.zeros((pad_e,), jnp.int32)])
    cols_p = jnp.concatenate([cols_i, jnp.zeros((pad_e,), jnp.int32)])
    mv_p = jnp.concatenate([mask_vals.astype(jnp.float32),
                            jnp.zeros((pad_e,), jnp.float32)])
    marks = jnp.arange(33, dtype=jnp.int32) * R_PER_
    bounds = jnp.searchsorted(rows_i, marks, side="left").astype(jnp.int32)
    bounds = jnp.concatenate([bounds, jnp.full((15,), E_, jnp.int32)])

    num, _den = _edge_sc(q, kv, rows_p, cols_p, mv_p, bounds)
    return _outproj(num, Wo, bo)


# speedup vs baseline: 1.0238x; 1.0238x over previous
import jax
import jax.numpy as jnp
import numpy as np
from jax import lax
from jax.experimental import pallas as pl
from jax.experimental.pallas import tpu as pltpu
from jax.experimental.pallas import tpu_sc as plsc

HID_ = 256
NH_ = 8
DH_ = HID_ // NH_
N_ = 10000
E_ = 160000

NW_ = 32
R_PER_ = 320
NPAD_ = NW_ * R_PER_
EB_ = 24
CH_ = 72
NSLOT_ = 3
EPAD_ = E_ + CH_
NRING_ = 8

_I32MIN = np.int32(-2147483648)


def _qkv_body(ht, hs, wq, wk, wv, bq, bk, bv, qo, kvo):
    dn = (((1,), (1,)), ((), ()))
    scale = jnp.float32(DH_ ** -0.5)
    qo[...] = (lax.dot_general(ht[...], wq[...], dn,
                               preferred_element_type=jnp.float32)
               + bq[...]) * scale
    kvo[:, 0:HID_] = lax.dot_general(hs[...], wk[...], dn,
                                     preferred_element_type=jnp.float32) + bk[...]
    kvo[:, HID_:2 * HID_] = lax.dot_general(hs[...], wv[...], dn,
                                            preferred_element_type=jnp.float32) + bv[...]


def _qkv(ht, hs, wq, wk, wv, bq, bk, bv):
    blk = 80
    grid = NPAD_ // blk
    nin = N_ // blk - 1
    in_spec = pl.BlockSpec((blk, HID_), lambda i: (jnp.minimum(i, nin), 0))
    row_spec = pl.BlockSpec((blk, HID_), lambda i: (i, 0))
    full_spec = pl.BlockSpec((HID_, HID_), lambda i: (0, 0))
    bias_spec = pl.BlockSpec((1, HID_), lambda i: (0, 0))
    return pl.pallas_call(
        _qkv_body,
        grid=(grid,),
        in_specs=[in_spec, in_spec, full_spec, full_spec, full_spec,
                  bias_spec, bias_spec, bias_spec],
        out_specs=[row_spec, pl.BlockSpec((blk, 2 * HID_), lambda i: (i, 0))],
        out_shape=[jax.ShapeDtypeStruct((NPAD_, HID_), jnp.float32),
                   jax.ShapeDtypeStruct((NPAD_, 2 * HID_), jnp.float32)],
    )(ht, hs, wq, wk, wv, bq.reshape(1, HID_), bk.reshape(1, HID_),
      bv.reshape(1, HID_))


def _lane_gather(vec, idx):
    dn = lax.GatherDimensionNumbers(offset_dims=(), collapsed_slice_dims=(0,),
                                    start_index_map=(0,))
    return lax.gather(vec, idx[:, None], dn, (1,),
                      mode=lax.GatherScatterMode.PROMISE_IN_BOUNDS)


def _extract_i32(vec, lane, iota):
    return jnp.max(jnp.where(iota == lane, vec, _I32MIN))


def _edge_body(q_hbm, kv_hbm, rows_hbm, cols_hbm, mv_hbm, bounds_hbm,
               num_hbm, den_hbm,
               bounds_v, rows_ch, cols_ch, mv_ch, kvb, qsl, den_acc, stg,
               sg, si, ss):
    c = lax.axis_index("c")
    s = lax.axis_index("s")
    wid = s * 2 + c
    iota = lax.iota(jnp.int32, 16)
    zero16 = jnp.zeros((16,), jnp.float32)
    NC16 = HID_ // 16

    pltpu.sync_copy(bounds_hbm, bounds_v)
    b0 = bounds_v[pl.ds(0, 16)]
    b1 = bounds_v[pl.ds(16, 16)]
    b2 = bounds_v[pl.ds(32, 16)]

    def bval(w):
        lane = w % 16
        grp = w // 16
        c0 = _extract_i32(b0, lane, iota)
        c1 = _extract_i32(b1, lane, iota)
        c2 = _extract_i32(b2, lane, iota)
        return jnp.where(grp == 0, c0, jnp.where(grp == 1, c1, c2))

    e_lo = bval(wid)
    e_hi = bval(wid + 1)
    r_lo = wid * R_PER_

    pltpu.sync_copy(q_hbm.at[pl.ds(r_lo, R_PER_)], qsl)

    def zrow(i, _):
        den_acc[i, pl.ds(0, 16)] = zero16
        return 0
    lax.fori_loop(0, R_PER_, zrow, 0)

    e0 = (e_lo // 8) * 8
    nch = (e_hi - e0 + CH_ - 1) // CH_
    npair = (nch + 1) // 2

    def fire_idx(ci, p):
        ec = pl.multiple_of(jnp.minimum(e0 + ci * CH_, EPAD_ - CH_), 8)
        pltpu.async_copy(rows_hbm.at[pl.ds(ec, CH_)], rows_ch.at[p], si.at[p])
        pltpu.async_copy(cols_hbm.at[pl.ds(ec, CH_)], cols_ch.at[p], si.at[p])
        pltpu.async_copy(mv_hbm.at[pl.ds(ec, CH_)], mv_ch.at[p], si.at[p])

    def wait_idx(p):
        pltpu.make_async_copy(rows_hbm.at[pl.ds(0, CH_)], rows_ch.at[p],
                              si.at[p]).wait()
        pltpu.make_async_copy(cols_hbm.at[pl.ds(0, CH_)], cols_ch.at[p],
                              si.at[p]).wait()
        pltpu.make_async_copy(mv_hbm.at[pl.ds(0, CH_)], mv_ch.at[p],
                              si.at[p]).wait()

    def fire_kv(p, k, slot):
        pltpu.async_copy(kv_hbm.at[cols_ch.at[p, pl.ds(k * EB_, EB_)]],
                         kvb.at[slot], sg.at[slot])

    def drain_kv(slot):
        pltpu.make_async_copy(kv_hbm.at[pl.ds(0, EB_)], kvb.at[slot],
                              sg.at[slot]).wait()

    def wait_one_store():
        pltpu.make_async_copy(stg.at[0], num_hbm.at[0], ss).wait()

    def do_flush(prev_rl, ns, accs, denv):
        owned = (prev_rl >= 0) & (prev_rl < R_PER_)

        @pl.when(owned & (ns >= NRING_))
        def _():
            wait_one_store()

        @pl.when(owned)
        def _():
            sl = ns % NRING_
            dsafe = jnp.where(denv == 0.0, jnp.float32(1.0), denv)
            for cc in range(NC16):
                stg[sl, pl.ds(cc * 16, 16)] = accs[cc] / dsafe
            pltpu.async_copy(stg.at[sl], num_hbm.at[r_lo + prev_rl], ss)
            psp = prev_rl + iota * 0
            plsc.addupdate_scatter(den_acc, [psp, iota], denv)
        return ns + jnp.where(owned, 1, 0)

    def block_compute(p, k, slot, ci, carry):
        base_blk = e0 + (ci * 3 + k) * EB_
        for jj, (off, jlo) in enumerate(((0, 0), (8, 8))):
            rows16 = rows_ch[p, pl.ds(k * EB_ + off, 16)]
            mv16 = mv_ch[p, pl.ds(k * EB_ + off, 16)]
            base_eg = base_blk + off

            def edge_body(j, ecarry, off=off, rows16=rows16, mv16=mv16,
                          base_eg=base_eg):
                prev_rl, ns, qregs, accs, denv = ecarry
                jsp = iota * 0 + j
                rowv = _lane_gather(rows16, jsp)
                mvalv = _lane_gather(mv16, jsp)
                rl = rowv[0] - r_lo
                changed = rl != prev_rl

                def on_change():
                    ns2 = do_flush(prev_rl, ns, accs, denv)
                    rlc = jnp.clip(rl, 0, R_PER_ - 1)
                    qn = tuple(qsl[rlc, pl.ds(cc * 16, 16)]
                               for cc in range(NC16))
                    return (ns2,) + qn + tuple(zero16 for _ in range(NC16)) \
                        + (zero16,)

                def no_change():
                    return (ns,) + tuple(qregs) + tuple(accs) + (denv,)

                st = lax.cond(changed, on_change, no_change)
                ns = st[0]
                qr = st[1:1 + NC16]
                ac = st[1 + NC16:1 + 2 * NC16]
                dv = st[1 + 2 * NC16]

                jrow = off + j
                parts = [qr[cc] * kvb[slot, jrow, pl.ds(cc * 16, 16)]
                         for cc in range(NC16)]
                while len(parts) > 1:
                    parts = [parts[2 * i] + parts[2 * i + 1]
                             for i in range(len(parts) // 2)]
                r = parts[0]
                fold = r + _lane_gather(r, jnp.bitwise_xor(iota, 8))
                eg = base_eg + j
                gate = jnp.where((eg >= e_lo) & (eg < e_hi),
                                 jnp.float32(1.0), jnp.float32(0.0))
                ex = jnp.exp(fold * mvalv) * gate
                ac = tuple(ac[cc] + ex * kvb[slot, jrow,
                                             pl.ds(HID_ + cc * 16, 16)]
                           for cc in range(NC16))
                return (rl, ns, qr, ac, dv + ex)

            carry = lax.fori_loop(jlo, 16, edge_body, carry)
        return carry

    def pair_body(g, carry):
        for p in range(2):
            ci = 2 * g + p
            for k in range(3):
                drain_kv(k)
                carry = block_compute(p, k, k, ci, carry)
                if k == 0:
                    fire_kv(p, 2, 2)
                elif k == 1:
                    wait_idx(1 - p)
                    fire_kv(1 - p, 0, 0)
                else:
                    fire_idx(ci + 2, p)
                    fire_kv(1 - p, 1, 1)
        return carry

    fire_idx(0, 0)
    fire_idx(1, 1)
    wait_idx(0)
    fire_kv(0, 0, 0)
    fire_kv(0, 1, 1)

    init = (jnp.int32(-(2 ** 30)), jnp.int32(0),
            tuple(zero16 for _ in range(NC16)),
            tuple(zero16 for _ in range(NC16)),
            zero16)
    prev_rl, ns, _, accs, denv = lax.fori_loop(0, npair, pair_body, init)
    ns = do_flush(prev_rl, ns, accs, denv)

    drain_kv(0)
    drain_kv(1)
    wait_idx(1)

    def dr(i, _):
        wait_one_store()
        return 0
    lax.fori_loop(0, jnp.minimum(ns, NRING_), dr, 0)

    for cc in range(NC16):
        stg[0, pl.ds(cc * 16, 16)] = zero16

    def zf(i, _):
        dv = den_acc[i, pl.ds(0, 16)]

        @pl.when(dv[0] == 0.0)
        def _():
            pltpu.sync_copy(stg.at[0], num_hbm.at[r_lo + i])
        return 0
    lax.fori_loop(0, R_PER_, zf, 0)

    pltpu.sync_copy(den_acc, den_hbm.at[pl.ds(r_lo, R_PER_)])


def _edge_sc(q, kv, rows_p, cols_p, mv_p, bounds):
    mesh = plsc.VectorSubcoreMesh(core_axis_name="c", subcore_axis_name="s")
    fn = pl.kernel(
        _edge_body,
        out_type=[jax.ShapeDtypeStruct((NPAD_, HID_), jnp.float32),
                  jax.ShapeDtypeStruct((NPAD_, 16), jnp.float32)],
        mesh=mesh,
        scratch_types=[
            pltpu.VMEM((48,), jnp.int32),
            pltpu.VMEM((2, CH_), jnp.int32),
            pltpu.VMEM((2, CH_), jnp.int32),
            pltpu.VMEM((2, CH_), jnp.float32),
            pltpu.VMEM((NSLOT_, EB_, 2 * HID_), jnp.float32),
            pltpu.VMEM((R_PER_, HID_), jnp.float32),
            pltpu.VMEM((R_PER_, 16), jnp.float32),
            pltpu.VMEM((NRING_, HID_), jnp.float32),
            pltpu.SemaphoreType.DMA((NSLOT_,)),
            pltpu.SemaphoreType.DMA((2,)),
            pltpu.SemaphoreType.DMA,
        ],
        compiler_params=pltpu.CompilerParams(needs_layout_passes=False,
                                             use_tc_tiling_on_sc=False),
    )
    return fn(q, kv, rows_p, cols_p, mv_p, bounds)


def _out_body(num, wo, bo, out):
    dn = (((1,), (1,)), ((), ()))
    out[...] = lax.dot_general(num[...], wo[...], dn,
                               preferred_element_type=jnp.float32) + bo[...]


def _outproj(num, wo, bo):
    blk = 200
    grid = N_ // blk
    return pl.pallas_call(
        _out_body,
        grid=(grid,),
        in_specs=[pl.BlockSpec((blk, HID_), lambda i: (i, 0)),
                  pl.BlockSpec((HID_, HID_), lambda i: (0, 0)),
                  pl.BlockSpec((1, HID_), lambda i: (0, 0))],
        out_specs=pl.BlockSpec((blk, HID_), lambda i: (i, 0)),
        out_shape=jax.ShapeDtypeStruct((N_, HID_), jnp.float32),
    )(num, wo, bo.reshape(1, HID_))


@jax.jit
def kernel(h_source, h_target, mask_rows, mask_cols, mask_vals,
           Wq, bq, Wk, bk, Wv, bv, Wo, bo):
    q, kv = _qkv(h_target, h_source, Wq, Wk, Wv, bq, bk, bv)

    rows_i = mask_rows.astype(jnp.int32)
    cols_i = mask_cols.astype(jnp.int32)
    pad_e = EPAD_ - E_
    rows_p = jnp.concatenate([rows_i, jnp.zeros((pad_e,), jnp.int32)])
    cols_p = jnp.concatenate([cols_i, jnp.zeros((pad_e,), jnp.int32)])
    mv_p = jnp.concatenate([mask_vals.astype(jnp.float32),
                            jnp.zeros((pad_e,), jnp.float32)])
    marks = jnp.arange(33, dtype=jnp.int32) * R_PER_
    bounds = jnp.searchsorted(rows_i, marks, side="left").astype(jnp.int32)
    bounds = jnp.concatenate([bounds, jnp.full((15,), E_, jnp.int32)])

    num, _den = _edge_sc(q, kv, rows_p, cols_p, mv_p, bounds)
    return _outproj(num, Wo, bo)
